# Initial kernel scaffold; baseline (speedup 1.0000x reference)
import jax
import jax.numpy as jnp
from jax.experimental import pallas as pl


def kernel(x, edge_index, batch, H1, post1, self1, b1, H2, post2, self2, b2, cW, cb):
    raise NotImplementedError("write your pallas kernel here")


# speedup vs baseline: 3.7643x; 3.7643x over previous
import functools

import jax
import jax.numpy as jnp
from jax import lax
from jax.experimental import pallas as pl
from jax.experimental.pallas import tpu as pltpu
from jax.experimental.pallas import tpu_sc as plsc

NCORE = 2
NSUB = 16
NW = NCORE * NSUB
KE = 128


def _build_agg(n_pad, d, blk, with_deg, interpret=False):
    rows_per_sub = n_pad // NSUB
    zr = 64
    assert rows_per_sub % zr == 0 and n_pad % (NSUB * 8) == 0

    mesh = plsc.VectorSubcoreMesh(
        core_axis_name="c", subcore_axis_name="s",
        num_cores=NCORE, num_subcores=NSUB)

    out_type = [jax.ShapeDtypeStruct((NCORE, n_pad, d), jnp.float32)]
    if with_deg:
        out_type.append(jax.ShapeDtypeStruct((NCORE, n_pad), jnp.float32))

    scratch = [
        pltpu.VMEM((KE,), jnp.int32),
        pltpu.VMEM((KE,), jnp.int32),
        pltpu.VMEM((KE, d), jnp.float32),
        pltpu.VMEM((KE,), jnp.float32),
        pltpu.VMEM((zr, d), jnp.float32),
        pltpu.VMEM_SHARED((n_pad, d), jnp.float32),
        pltpu.VMEM_SHARED((n_pad,), jnp.float32),
        pltpu.SemaphoreType.DMA,
    ]

    def body(h_hbm, dst_hbm, src_hbm, *rest):
        if with_deg:
            p_out, deg_out = rest[0], rest[1]
            rest = rest[2:]
        else:
            p_out = rest[0]
            rest = rest[1:]
        dst_v, src_v, rows_v, ones_v, zbuf, acc, dacc, sem = rest

        cid = lax.axis_index("c")
        sid = lax.axis_index("s")
        wid = sid * NCORE + cid

        z16 = jnp.zeros((16,), jnp.float32)
        o16 = jnp.ones((16,), jnp.float32)

        def fill_row(i, _):
            for j in range(d // 16):
                zbuf[i, pl.ds(j * 16, 16)] = z16
            return 0
        lax.fori_loop(0, zr, fill_row, 0)

        def fill_ones(i, _):
            ones_v[pl.ds(i * 16, 16)] = o16
            return 0
        lax.fori_loop(0, KE // 16, fill_ones, 0)

        def clear_blk(k, _):
            pltpu.sync_copy(zbuf, acc.at[pl.ds(sid * rows_per_sub + k * zr, zr)])
            return 0
        lax.fori_loop(0, rows_per_sub // zr, clear_blk, 0)
        if with_deg:
            def clear_deg(k, _):
                pltpu.sync_copy(zbuf.at[0],
                                dacc.at[pl.ds(sid * rows_per_sub + k * d, d)])
                return 0
            lax.fori_loop(0, rows_per_sub // d, clear_deg, 0)
        plsc.subcore_barrier()

        def edge_blk(b, _):
            pltpu.sync_copy(dst_hbm.at[wid, b], dst_v)
            pltpu.async_copy(h_hbm.at[dst_v], rows_v, sem).wait()
            pltpu.sync_copy(src_hbm.at[wid, b], src_v)
            pltpu.sync_copy(rows_v, acc.at[src_v], add=True)
            if with_deg:
                pltpu.sync_copy(ones_v, dacc.at[src_v], add=True)
            return 0
        lax.fori_loop(0, blk, edge_blk, 0)
        plsc.subcore_barrier()

        base = sid * rows_per_sub
        pltpu.sync_copy(acc.at[pl.ds(base, rows_per_sub)],
                        p_out.at[cid, pl.ds(base, rows_per_sub)])
        if with_deg:
            pltpu.sync_copy(dacc.at[pl.ds(base, rows_per_sub)],
                            deg_out.at[cid, pl.ds(base, rows_per_sub)])

    return pl.kernel(body, out_type=out_type, mesh=mesh,
                     scratch_types=scratch, interpret=interpret)


def _dense_layer(p, h, deg_col, hcat, post_t, self_t, b_row, interpret=False):
    n_pad, d = h.shape
    dh = hcat.shape[1]
    r = 1024
    assert n_pad % r == 0
    grid = (n_pad // r,)

    def body(p_ref, h_ref, deg_ref, hcat_ref, post_ref, self_ref, b_ref, o_ref):
        ssum = p_ref[0] + p_ref[1]
        hv = h_ref[...]
        deg = deg_ref[...]
        base = jnp.where(deg > 0.0, ssum / jnp.maximum(deg, 1.0), hv)
        hashed = jnp.dot(base, hcat_ref[...], preferred_element_type=jnp.float32)
        out = jnp.dot(hashed, post_ref[...], preferred_element_type=jnp.float32)
        out = out + jnp.dot(hv, self_ref[...], preferred_element_type=jnp.float32)
        out = out + b_ref[...]
        o_ref[...] = jnp.where(out > 0.0, out, jnp.exp(out) - 1.0)

    return pl.pallas_call(
        body,
        grid=grid,
        in_specs=[
            pl.BlockSpec((NCORE, r, d), lambda i: (0, i, 0)),
            pl.BlockSpec((r, d), lambda i: (i, 0)),
            pl.BlockSpec((r, 1), lambda i: (i, 0)),
            pl.BlockSpec((d, dh), lambda i: (0, 0)),
            pl.BlockSpec((dh, d), lambda i: (0, 0)),
            pl.BlockSpec((d, d), lambda i: (0, 0)),
            pl.BlockSpec((1, d), lambda i: (0, 0)),
        ],
        out_specs=pl.BlockSpec((r, d), lambda i: (i, 0)),
        out_shape=jax.ShapeDtypeStruct((n_pad, d), jnp.float32),
        interpret=interpret,
    )(p, h, deg_col, hcat, post_t, self_t, b_row)


def _pool_classify(h, batch3, cw_t, cb_row, n_graphs, interpret=False):
    n_pad, d = h.shape
    pb, _, pr = batch3.shape
    nc = cw_t.shape[1]
    grid = (pb,)

    def body(h_ref, b_ref, cw_ref, cb_ref, o_ref, gacc, cacc):
        i = pl.program_id(0)

        @pl.when(i == 0)
        def _():
            gacc[...] = jnp.zeros_like(gacc)
            cacc[...] = jnp.zeros_like(cacc)

        ids = b_ref[0]
        gids = lax.broadcasted_iota(jnp.int32, (n_graphs, pr), 0)
        onehot = (ids == gids).astype(jnp.float32)
        gacc[...] += jnp.dot(onehot, h_ref[...],
                             preferred_element_type=jnp.float32)
        cacc[...] += jnp.sum(onehot, axis=1, keepdims=True)

        @pl.when(i == pb - 1)
        def _():
            g = gacc[...] / jnp.maximum(cacc[...], 1.0)
            o_ref[...] = jnp.dot(g, cw_ref[...],
                                 preferred_element_type=jnp.float32) + cb_ref[...]

    return pl.pallas_call(
        body,
        grid=grid,
        in_specs=[
            pl.BlockSpec((pr, d), lambda i: (i, 0)),
            pl.BlockSpec((1, 1, pr), lambda i: (i, 0, 0)),
            pl.BlockSpec((d, nc), lambda i: (0, 0)),
            pl.BlockSpec((1, nc), lambda i: (0, 0)),
        ],
        out_specs=pl.BlockSpec((n_graphs, nc), lambda i: (0, 0)),
        out_shape=jax.ShapeDtypeStruct((n_graphs, nc), jnp.float32),
        scratch_shapes=[
            pltpu.VMEM((n_graphs, d), jnp.float32),
            pltpu.VMEM((n_graphs, 1), jnp.float32),
        ],
        interpret=interpret,
    )(h, batch3, cw_t, cb_row)


def kernel(x, edge_index, batch, H1, post1, self1, b1, H2, post2, self2, b2,
           cW, cb):
    n, d = x.shape
    e = edge_index.shape[1]
    g = 64
    nh = H1.shape[0]

    n_pad = -(-(n + 1) // (NSUB * 8 * 2)) * (NSUB * 8 * 2)
    if n_pad % 1024 != 0:
        n_pad = -(-n_pad // 1024) * 1024

    src = edge_index[0]
    dst = edge_index[1]
    blk = -(-e // (NW * KE))
    e_pad = NW * blk * KE
    dst_p = jnp.concatenate(
        [dst, jnp.zeros((e_pad - e,), dst.dtype)]).reshape(NW, blk, KE)
    src_p = jnp.concatenate(
        [src, jnp.full((e_pad - e,), n, src.dtype)]).reshape(NW, blk, KE)

    x_pad = jnp.concatenate(
        [x, jnp.zeros((n_pad - n, d), jnp.float32)], axis=0)

    hcat1 = jnp.transpose(H1, (1, 0, 2)).reshape(d, nh * H1.shape[2])
    hcat2 = jnp.transpose(H2, (1, 0, 2)).reshape(d, nh * H2.shape[2])

    agg_deg = _build_agg(n_pad, d, blk, with_deg=True)
    agg = _build_agg(n_pad, d, blk, with_deg=False)

    p1, deg_p = agg_deg(x_pad, dst_p, src_p)
    deg_col = (deg_p[0] + deg_p[1])[:, None]

    h1 = _dense_layer(p1, x_pad, deg_col, hcat1, post1.T, self1.T, b1[None, :])

    (p2,) = agg(h1, dst_p, src_p)
    h2 = _dense_layer(p2, h1, deg_col, hcat2, post2.T, self2.T, b2[None, :])

    pr = 2000
    pb = n // pr
    batch3 = batch.reshape(pb, 1, pr)
    return _pool_classify(h2, batch3, cW.T, cb[None, :], g)


# speedup vs baseline: 4.9195x; 1.3069x over previous
import functools

import jax
import jax.numpy as jnp
from jax import lax
from jax.experimental import pallas as pl
from jax.experimental.pallas import tpu as pltpu
from jax.experimental.pallas import tpu_sc as plsc

NCORE = 2
NSUB = 16
NW = NCORE * NSUB
KE = 128


def _build_agg(n_pad, d, blk, with_deg, interpret=False):
    rows_per_sub = n_pad // NSUB
    zr = 16
    assert rows_per_sub % zr == 0 and n_pad % (NSUB * 8) == 0

    mesh = plsc.VectorSubcoreMesh(
        core_axis_name="c", subcore_axis_name="s",
        num_cores=NCORE, num_subcores=NSUB)

    out_type = [jax.ShapeDtypeStruct((NCORE, n_pad, d), jnp.float32)]
    if with_deg:
        out_type.append(jax.ShapeDtypeStruct((NCORE, n_pad), jnp.float32))

    scratch = [
        pltpu.VMEM((2, KE), jnp.int32),
        pltpu.VMEM((2, KE), jnp.int32),
        pltpu.VMEM((2, KE, d), jnp.float32),
        pltpu.VMEM((KE,), jnp.float32),
        pltpu.VMEM((zr, d), jnp.float32),
        pltpu.VMEM_SHARED((n_pad, d), jnp.float32),
        pltpu.VMEM_SHARED((n_pad,), jnp.float32),
        pltpu.SemaphoreType.DMA,
        pltpu.SemaphoreType.DMA,
    ]

    def body(h_hbm, dst_hbm, src_hbm, *rest):
        if with_deg:
            p_out, deg_out = rest[0], rest[1]
            rest = rest[2:]
        else:
            p_out = rest[0]
            rest = rest[1:]
        dst_v, src_v, rows_v, ones_v, zbuf, acc, dacc, isem, gsem = rest

        cid = lax.axis_index("c")
        sid = lax.axis_index("s")
        wid = sid * NCORE + cid

        z16 = jnp.zeros((16,), jnp.float32)
        o16 = jnp.ones((16,), jnp.float32)

        idx_dma = pltpu.async_copy(dst_hbm.at[wid, 0], dst_v.at[0], isem)
        idx_dma2 = pltpu.async_copy(src_hbm.at[wid, 0], src_v.at[0], isem)

        def fill_row(i, _):
            for j in range(d // 16):
                zbuf[i, pl.ds(j * 16, 16)] = z16
            return 0
        lax.fori_loop(0, zr, fill_row, 0)

        def fill_ones(i, _):
            ones_v[pl.ds(i * 16, 16)] = o16
            return 0
        lax.fori_loop(0, KE // 16, fill_ones, 0)

        def clear_blk(k, _):
            pltpu.sync_copy(zbuf, acc.at[pl.ds(sid * rows_per_sub + k * zr, zr)])
            return 0
        lax.fori_loop(0, rows_per_sub // zr, clear_blk, 0)
        if with_deg:
            def clear_deg(k, _):
                pltpu.sync_copy(zbuf.at[0],
                                dacc.at[pl.ds(sid * rows_per_sub + k * d, d)])
                return 0
            lax.fori_loop(0, rows_per_sub // d, clear_deg, 0)
        idx_dma.wait()
        idx_dma2.wait()
        plsc.subcore_barrier()

        def edge_blk(b, _):
            p = lax.rem(b, 2)
            q = lax.rem(b + 1, 2)

            @pl.when(b >= 1)
            def _():
                pltpu.make_async_copy(dst_hbm.at[wid, b], dst_v.at[p],
                                      isem).wait()
                pltpu.make_async_copy(src_hbm.at[wid, b], src_v.at[p],
                                      isem).wait()
            g = pltpu.async_copy(h_hbm.at[dst_v.at[p]], rows_v.at[p], gsem)

            @pl.when(b >= 1)
            def _():
                pltpu.sync_copy(rows_v.at[q], acc.at[src_v.at[q]], add=True)
                if with_deg:
                    pltpu.sync_copy(ones_v, dacc.at[src_v.at[q]], add=True)

            @pl.when(b + 1 < blk)
            def _():
                pltpu.async_copy(dst_hbm.at[wid, b + 1], dst_v.at[q], isem)
                pltpu.async_copy(src_hbm.at[wid, b + 1], src_v.at[q], isem)
            g.wait()
            return 0
        lax.fori_loop(0, blk, edge_blk, 0)
        qlast = (blk - 1) % 2
        pltpu.sync_copy(rows_v.at[qlast], acc.at[src_v.at[qlast]], add=True)
        if with_deg:
            pltpu.sync_copy(ones_v, dacc.at[src_v.at[qlast]], add=True)
        plsc.subcore_barrier()

        base = sid * rows_per_sub
        pltpu.sync_copy(acc.at[pl.ds(base, rows_per_sub)],
                        p_out.at[cid, pl.ds(base, rows_per_sub)])
        if with_deg:
            pltpu.sync_copy(dacc.at[pl.ds(base, rows_per_sub)],
                            deg_out.at[cid, pl.ds(base, rows_per_sub)])

    return pl.kernel(body, out_type=out_type, mesh=mesh,
                     scratch_types=scratch, interpret=interpret)


def _dense_layer(p, h, deg_col, hcat, post_t, self_t, b_row, interpret=False):
    n_pad, d = h.shape
    dh = hcat.shape[1]
    r = 1024
    assert n_pad % r == 0
    grid = (n_pad // r,)

    def body(p_ref, h_ref, deg_ref, hcat_ref, post_ref, self_ref, b_ref, o_ref):
        ssum = p_ref[0] + p_ref[1]
        hv = h_ref[...]
        deg = deg_ref[...]
        base = jnp.where(deg > 0.0, ssum / jnp.maximum(deg, 1.0), hv)
        hashed = jnp.dot(base, hcat_ref[...], preferred_element_type=jnp.float32)
        out = jnp.dot(hashed, post_ref[...], preferred_element_type=jnp.float32)
        out = out + jnp.dot(hv, self_ref[...], preferred_element_type=jnp.float32)
        out = out + b_ref[...]
        o_ref[...] = jnp.where(out > 0.0, out, jnp.exp(out) - 1.0)

    return pl.pallas_call(
        body,
        grid=grid,
        in_specs=[
            pl.BlockSpec((NCORE, r, d), lambda i: (0, i, 0)),
            pl.BlockSpec((r, d), lambda i: (i, 0)),
            pl.BlockSpec((r, 1), lambda i: (i, 0)),
            pl.BlockSpec((d, dh), lambda i: (0, 0)),
            pl.BlockSpec((dh, d), lambda i: (0, 0)),
            pl.BlockSpec((d, d), lambda i: (0, 0)),
            pl.BlockSpec((1, d), lambda i: (0, 0)),
        ],
        out_specs=pl.BlockSpec((r, d), lambda i: (i, 0)),
        out_shape=jax.ShapeDtypeStruct((n_pad, d), jnp.float32),
        interpret=interpret,
    )(p, h, deg_col, hcat, post_t, self_t, b_row)


def _pool_classify(h, batch3, cw_t, cb_row, n_graphs, interpret=False):
    n_pad, d = h.shape
    pb, _, pr = batch3.shape
    nc = cw_t.shape[1]
    grid = (pb,)

    def body(h_ref, b_ref, cw_ref, cb_ref, o_ref, gacc, cacc):
        i = pl.program_id(0)

        @pl.when(i == 0)
        def _():
            gacc[...] = jnp.zeros_like(gacc)
            cacc[...] = jnp.zeros_like(cacc)

        ids = b_ref[0]
        gids = lax.broadcasted_iota(jnp.int32, (n_graphs, pr), 0)
        onehot = (ids == gids).astype(jnp.float32)
        gacc[...] += jnp.dot(onehot, h_ref[...],
                             preferred_element_type=jnp.float32)
        cacc[...] += jnp.sum(onehot, axis=1, keepdims=True)

        @pl.when(i == pb - 1)
        def _():
            g = gacc[...] / jnp.maximum(cacc[...], 1.0)
            o_ref[...] = jnp.dot(g, cw_ref[...],
                                 preferred_element_type=jnp.float32) + cb_ref[...]

    return pl.pallas_call(
        body,
        grid=grid,
        in_specs=[
            pl.BlockSpec((pr, d), lambda i: (i, 0)),
            pl.BlockSpec((1, 1, pr), lambda i: (i, 0, 0)),
            pl.BlockSpec((d, nc), lambda i: (0, 0)),
            pl.BlockSpec((1, nc), lambda i: (0, 0)),
        ],
        out_specs=pl.BlockSpec((n_graphs, nc), lambda i: (0, 0)),
        out_shape=jax.ShapeDtypeStruct((n_graphs, nc), jnp.float32),
        scratch_shapes=[
            pltpu.VMEM((n_graphs, d), jnp.float32),
            pltpu.VMEM((n_graphs, 1), jnp.float32),
        ],
        interpret=interpret,
    )(h, batch3, cw_t, cb_row)


def kernel(x, edge_index, batch, H1, post1, self1, b1, H2, post2, self2, b2,
           cW, cb):
    n, d = x.shape
    e = edge_index.shape[1]
    g = 64
    nh = H1.shape[0]

    n_pad = -(-(n + 1) // (NSUB * 8 * 2)) * (NSUB * 8 * 2)
    if n_pad % 1024 != 0:
        n_pad = -(-n_pad // 1024) * 1024

    src = edge_index[0]
    dst = edge_index[1]
    blk = -(-e // (NW * KE))
    e_pad = NW * blk * KE
    dst_p = jnp.concatenate(
        [dst, jnp.zeros((e_pad - e,), dst.dtype)]).reshape(NW, blk, KE)
    src_p = jnp.concatenate(
        [src, jnp.full((e_pad - e,), n, src.dtype)]).reshape(NW, blk, KE)

    x_pad = jnp.concatenate(
        [x, jnp.zeros((n_pad - n, d), jnp.float32)], axis=0)

    hcat1 = jnp.transpose(H1, (1, 0, 2)).reshape(d, nh * H1.shape[2])
    hcat2 = jnp.transpose(H2, (1, 0, 2)).reshape(d, nh * H2.shape[2])

    agg_deg = _build_agg(n_pad, d, blk, with_deg=True)
    agg = _build_agg(n_pad, d, blk, with_deg=False)

    p1, deg_p = agg_deg(x_pad, dst_p, src_p)
    deg_col = (deg_p[0] + deg_p[1])[:, None]

    h1 = _dense_layer(p1, x_pad, deg_col, hcat1, post1.T, self1.T, b1[None, :])

    (p2,) = agg(h1, dst_p, src_p)
    h2 = _dense_layer(p2, h1, deg_col, hcat2, post2.T, self2.T, b2[None, :])

    pr = 2000
    pb = n // pr
    batch3 = batch.reshape(pb, 1, pr)
    return _pool_classify(h2, batch3, cW.T, cb[None, :], g)


# speedup vs baseline: 13.0215x; 2.6469x over previous
import functools

import jax
import jax.numpy as jnp
from jax import lax
from jax.experimental import pallas as pl
from jax.experimental.pallas import tpu as pltpu
from jax.experimental.pallas import tpu_sc as plsc

NCORE = 2
NSUB = 16
NW = NCORE * NSUB
KE = 80
NRB = 4
NIB = 8
ILAG = 3
GLAG = 3
SLAG = 4


def _build_agg(n_pad, d, blk, with_deg, interpret=False):
    rows_per_sub = n_pad // NSUB
    zr = 16
    assert rows_per_sub % zr == 0 and n_pad % (NSUB * 8) == 0
    assert blk > SLAG + 1

    mesh = plsc.VectorSubcoreMesh(
        core_axis_name="c", subcore_axis_name="s",
        num_cores=NCORE, num_subcores=NSUB)

    out_type = [jax.ShapeDtypeStruct((NCORE, n_pad, d), jnp.float32)]
    if with_deg:
        out_type.append(jax.ShapeDtypeStruct((NCORE, n_pad), jnp.float32))

    scratch = [
        pltpu.VMEM((NIB, 2, KE), jnp.int32),
        pltpu.VMEM((NRB, KE, d), jnp.float32),
        pltpu.VMEM((KE,), jnp.float32),
        pltpu.VMEM((zr, d), jnp.float32),
        pltpu.VMEM_SHARED((n_pad, d), jnp.float32),
        pltpu.VMEM_SHARED((n_pad,), jnp.float32),
        pltpu.SemaphoreType.DMA,
        pltpu.SemaphoreType.DMA,
        pltpu.SemaphoreType.DMA,
        pltpu.SemaphoreType.DMA,
    ]

    def body(h_hbm, idx_hbm, *rest):
        if with_deg:
            p_out, deg_out = rest[0], rest[1]
            rest = rest[2:]
        else:
            p_out = rest[0]
            rest = rest[1:]
        ibuf, rows_v, ones_v, zbuf, acc, dacc, isem, gsem, ssem, osem = rest

        cid = lax.axis_index("c")
        sid = lax.axis_index("s")
        wid = sid * NCORE + cid

        z16 = jnp.zeros((16,), jnp.float32)
        o16 = jnp.ones((16,), jnp.float32)

        def idx_desc(b):
            return pltpu.make_async_copy(
                idx_hbm.at[wid, b], ibuf.at[lax.rem(b, NIB)], isem)

        def gather_desc(b):
            return pltpu.make_async_copy(
                h_hbm.at[ibuf.at[lax.rem(b, NIB), 0]],
                rows_v.at[lax.rem(b, NRB)], gsem)

        def scat_desc(b):
            return pltpu.make_async_copy(
                rows_v.at[lax.rem(b, NRB)],
                acc.at[ibuf.at[lax.rem(b, NIB), 1]], ssem)

        def ones_desc(b):
            return pltpu.make_async_copy(
                ones_v, dacc.at[ibuf.at[lax.rem(b, NIB), 1]], osem)

        for t in range(ILAG):
            idx_desc(t).start()

        def fill_row(i, _):
            for j in range(d // 16):
                zbuf[i, pl.ds(j * 16, 16)] = z16
            return 0
        lax.fori_loop(0, zr, fill_row, 0)

        def fill_ones(i, _):
            ones_v[pl.ds(i * 16, 16)] = o16
            return 0
        lax.fori_loop(0, KE // 16, fill_ones, 0)

        def clear_blk(k, _):
            pltpu.sync_copy(zbuf, acc.at[pl.ds(sid * rows_per_sub + k * zr, zr)])
            return 0
        lax.fori_loop(0, rows_per_sub // zr, clear_blk, 0)
        if with_deg:
            def clear_deg(k, _):
                pltpu.sync_copy(zbuf.at[0],
                                dacc.at[pl.ds(sid * rows_per_sub + k * d, d)])
                return 0
            lax.fori_loop(0, rows_per_sub // d, clear_deg, 0)
        plsc.subcore_barrier()

        def edge_blk(b, _):
            @pl.when(b >= SLAG)
            def _():
                scat_desc(b - SLAG).wait()
                if with_deg:
                    ones_desc(b - SLAG).wait()
            idx_desc(b).wait()
            gather_desc(b).start()

            @pl.when(b >= GLAG)
            def _():
                gather_desc(b - GLAG).wait()
                scat_desc(b - GLAG).start(add=True)
                if with_deg:
                    ones_desc(b - GLAG).start(add=True)

            @pl.when(b + ILAG < blk)
            def _():
                idx_desc(b + ILAG).start()
            return 0
        lax.fori_loop(0, blk, edge_blk, 0)

        for t in range(blk - GLAG, blk):
            gather_desc(t).wait()
            scat_desc(t).start(add=True)
            if with_deg:
                ones_desc(t).start(add=True)
        for t in range(blk - SLAG, blk):
            scat_desc(t).wait()
            if with_deg:
                ones_desc(t).wait()
        plsc.subcore_barrier()

        base = sid * rows_per_sub
        pltpu.sync_copy(acc.at[pl.ds(base, rows_per_sub)],
                        p_out.at[cid, pl.ds(base, rows_per_sub)])
        if with_deg:
            pltpu.sync_copy(dacc.at[pl.ds(base, rows_per_sub)],
                            deg_out.at[cid, pl.ds(base, rows_per_sub)])

    return pl.kernel(body, out_type=out_type, mesh=mesh,
                     scratch_types=scratch, interpret=interpret)


def _dense_layer(p, h, deg_col, hcat, post_t, self_t, b_row, interpret=False):
    n_pad, d = h.shape
    dh = hcat.shape[1]
    r = 1024
    assert n_pad % r == 0
    grid = (n_pad // r,)

    def body(p_ref, h_ref, deg_ref, hcat_ref, post_ref, self_ref, b_ref, o_ref):
        ssum = p_ref[0] + p_ref[1]
        hv = h_ref[...]
        deg = deg_ref[...]
        base = jnp.where(deg > 0.0, ssum / jnp.maximum(deg, 1.0), hv)
        hashed = jnp.dot(base, hcat_ref[...], preferred_element_type=jnp.float32)
        out = jnp.dot(hashed, post_ref[...], preferred_element_type=jnp.float32)
        out = out + jnp.dot(hv, self_ref[...], preferred_element_type=jnp.float32)
        out = out + b_ref[...]
        o_ref[...] = jnp.where(out > 0.0, out, jnp.exp(out) - 1.0)

    return pl.pallas_call(
        body,
        grid=grid,
        in_specs=[
            pl.BlockSpec((NCORE, r, d), lambda i: (0, i, 0)),
            pl.BlockSpec((r, d), lambda i: (i, 0)),
            pl.BlockSpec((r, 1), lambda i: (i, 0)),
            pl.BlockSpec((d, dh), lambda i: (0, 0)),
            pl.BlockSpec((dh, d), lambda i: (0, 0)),
            pl.BlockSpec((d, d), lambda i: (0, 0)),
            pl.BlockSpec((1, d), lambda i: (0, 0)),
        ],
        out_specs=pl.BlockSpec((r, d), lambda i: (i, 0)),
        out_shape=jax.ShapeDtypeStruct((n_pad, d), jnp.float32),
        interpret=interpret,
    )(p, h, deg_col, hcat, post_t, self_t, b_row)


def _pool_classify(h, batch3, cw_t, cb_row, n_graphs, interpret=False):
    n_pad, d = h.shape
    pb, _, pr = batch3.shape
    nc = cw_t.shape[1]
    grid = (pb,)

    def body(h_ref, b_ref, cw_ref, cb_ref, o_ref, gacc, cacc):
        i = pl.program_id(0)

        @pl.when(i == 0)
        def _():
            gacc[...] = jnp.zeros_like(gacc)
            cacc[...] = jnp.zeros_like(cacc)

        ids = b_ref[0]
        gids = lax.broadcasted_iota(jnp.int32, (n_graphs, pr), 0)
        onehot = (ids == gids).astype(jnp.float32)
        gacc[...] += jnp.dot(onehot, h_ref[...],
                             preferred_element_type=jnp.float32)
        cacc[...] += jnp.sum(onehot, axis=1, keepdims=True)

        @pl.when(i == pb - 1)
        def _():
            g = gacc[...] / jnp.maximum(cacc[...], 1.0)
            o_ref[...] = jnp.dot(g, cw_ref[...],
                                 preferred_element_type=jnp.float32) + cb_ref[...]

    return pl.pallas_call(
        body,
        grid=grid,
        in_specs=[
            pl.BlockSpec((pr, d), lambda i: (i, 0)),
            pl.BlockSpec((1, 1, pr), lambda i: (i, 0, 0)),
            pl.BlockSpec((d, nc), lambda i: (0, 0)),
            pl.BlockSpec((1, nc), lambda i: (0, 0)),
        ],
        out_specs=pl.BlockSpec((n_graphs, nc), lambda i: (0, 0)),
        out_shape=jax.ShapeDtypeStruct((n_graphs, nc), jnp.float32),
        scratch_shapes=[
            pltpu.VMEM((n_graphs, d), jnp.float32),
            pltpu.VMEM((n_graphs, 1), jnp.float32),
        ],
        interpret=interpret,
    )(h, batch3, cw_t, cb_row)


def kernel(x, edge_index, batch, H1, post1, self1, b1, H2, post2, self2, b2,
           cW, cb):
    n, d = x.shape
    e = edge_index.shape[1]
    g = 64
    nh = H1.shape[0]

    n_pad = -(-(n + 1) // (NSUB * 8 * 2)) * (NSUB * 8 * 2)
    if n_pad % 1024 != 0:
        n_pad = -(-n_pad // 1024) * 1024

    src = edge_index[0]
    dst = edge_index[1]
    blk = -(-e // (NW * KE))
    e_pad = NW * blk * KE
    dst_p = jnp.concatenate(
        [dst, jnp.zeros((e_pad - e,), dst.dtype)]).reshape(NW, blk, KE)
    src_p = jnp.concatenate(
        [src, jnp.full((e_pad - e,), n, src.dtype)]).reshape(NW, blk, KE)
    idx_p = jnp.stack([dst_p, src_p], axis=2)

    x_pad = jnp.concatenate(
        [x, jnp.zeros((n_pad - n, d), jnp.float32)], axis=0)

    hcat1 = jnp.transpose(H1, (1, 0, 2)).reshape(d, nh * H1.shape[2])
    hcat2 = jnp.transpose(H2, (1, 0, 2)).reshape(d, nh * H2.shape[2])

    agg_deg = _build_agg(n_pad, d, blk, with_deg=True)
    agg = _build_agg(n_pad, d, blk, with_deg=False)

    p1, deg_p = agg_deg(x_pad, idx_p)
    deg_col = (deg_p[0] + deg_p[1])[:, None]

    h1 = _dense_layer(p1, x_pad, deg_col, hcat1, post1.T, self1.T, b1[None, :])

    (p2,) = agg(h1, idx_p)
    h2 = _dense_layer(p2, h1, deg_col, hcat2, post2.T, self2.T, b2[None, :])

    pr = 2000
    pb = n // pr
    batch3 = batch.reshape(pb, 1, pr)
    return _pool_classify(h2, batch3, cW.T, cb[None, :], g)


# speedup vs baseline: 13.6925x; 1.0515x over previous
import functools

import jax
import jax.numpy as jnp
from jax import lax
from jax.experimental import pallas as pl
from jax.experimental.pallas import tpu as pltpu
from jax.experimental.pallas import tpu_sc as plsc

NCORE = 2
NSUB = 16
NW = NCORE * NSUB
KE = 80
NRB = 4
NIB = 8
ILAG = 3
GLAG = 3
SLAG = 4


def _build_agg(n_pad, d, blk, with_deg, interpret=False):
    rows_per_sub = n_pad // NSUB
    zr = 16
    assert rows_per_sub % zr == 0 and n_pad % (NSUB * 8) == 0
    assert blk > SLAG + 1

    mesh = plsc.VectorSubcoreMesh(
        core_axis_name="c", subcore_axis_name="s",
        num_cores=NCORE, num_subcores=NSUB)

    out_type = [jax.ShapeDtypeStruct((NCORE, n_pad, d), jnp.float32)]
    if with_deg:
        out_type.append(jax.ShapeDtypeStruct((NCORE, n_pad), jnp.float32))

    scratch = [
        pltpu.VMEM((NIB, 2, KE), jnp.int32),
        pltpu.VMEM((NRB, KE, d), jnp.float32),
        pltpu.VMEM((KE,), jnp.float32),
        pltpu.VMEM((zr, d), jnp.float32),
        pltpu.VMEM_SHARED((n_pad, d), jnp.float32),
        pltpu.VMEM_SHARED((n_pad,), jnp.float32),
        pltpu.SemaphoreType.DMA,
        pltpu.SemaphoreType.DMA,
        pltpu.SemaphoreType.DMA,
        pltpu.SemaphoreType.DMA,
    ]

    def body(h_hbm, idx_hbm, *rest):
        if with_deg:
            p_out, deg_out = rest[0], rest[1]
            rest = rest[2:]
        else:
            p_out = rest[0]
            rest = rest[1:]
        ibuf, rows_v, ones_v, zbuf, acc, dacc, isem, gsem, ssem, osem = rest

        cid = lax.axis_index("c")
        sid = lax.axis_index("s")
        wid = sid * NCORE + cid

        z16 = jnp.zeros((16,), jnp.float32)
        o16 = jnp.ones((16,), jnp.float32)

        def idx_desc(b):
            return pltpu.make_async_copy(
                idx_hbm.at[wid, b], ibuf.at[lax.rem(b, NIB)], isem)

        def gather_desc(b):
            return pltpu.make_async_copy(
                h_hbm.at[ibuf.at[lax.rem(b, NIB), 0]],
                rows_v.at[lax.rem(b, NRB)], gsem)

        def scat_desc(b):
            return pltpu.make_async_copy(
                rows_v.at[lax.rem(b, NRB)],
                acc.at[ibuf.at[lax.rem(b, NIB), 1]], ssem)

        def ones_desc(b):
            return pltpu.make_async_copy(
                ones_v, dacc.at[ibuf.at[lax.rem(b, NIB), 1]], osem)

        for t in range(ILAG):
            idx_desc(t).start()

        def fill_row(i, _):
            for j in range(d // 16):
                zbuf[i, pl.ds(j * 16, 16)] = z16
            return 0
        lax.fori_loop(0, zr, fill_row, 0)

        def fill_ones(i, _):
            ones_v[pl.ds(i * 16, 16)] = o16
            return 0
        lax.fori_loop(0, KE // 16, fill_ones, 0)

        def clear_blk(k, _):
            pltpu.async_copy(
                zbuf, acc.at[pl.ds(sid * rows_per_sub + k * zr, zr)], gsem)
            return 0
        lax.fori_loop(0, rows_per_sub // zr, clear_blk, 0)
        if with_deg:
            def clear_deg(k, _):
                pltpu.async_copy(
                    zbuf.at[0],
                    dacc.at[pl.ds(sid * rows_per_sub + k * d, d)], osem)
                return 0
            lax.fori_loop(0, rows_per_sub // d, clear_deg, 0)

        def drain_blk(k, _):
            pltpu.make_async_copy(
                zbuf, acc.at[pl.ds(sid * rows_per_sub + k * zr, zr)],
                gsem).wait()
            return 0
        lax.fori_loop(0, rows_per_sub // zr, drain_blk, 0)
        if with_deg:
            def drain_deg(k, _):
                pltpu.make_async_copy(
                    zbuf.at[0],
                    dacc.at[pl.ds(sid * rows_per_sub + k * d, d)],
                    osem).wait()
                return 0
            lax.fori_loop(0, rows_per_sub // d, drain_deg, 0)
        plsc.subcore_barrier()

        def edge_blk(b, _):
            @pl.when(b >= SLAG)
            def _():
                scat_desc(b - SLAG).wait()
                if with_deg:
                    ones_desc(b - SLAG).wait()
            idx_desc(b).wait()
            gather_desc(b).start()

            @pl.when(b >= GLAG)
            def _():
                gather_desc(b - GLAG).wait()
                scat_desc(b - GLAG).start(add=True)
                if with_deg:
                    ones_desc(b - GLAG).start(add=True)

            @pl.when(b + ILAG < blk)
            def _():
                idx_desc(b + ILAG).start()
            return 0
        lax.fori_loop(0, blk, edge_blk, 0)

        for t in range(blk - GLAG, blk):
            gather_desc(t).wait()
            scat_desc(t).start(add=True)
            if with_deg:
                ones_desc(t).start(add=True)
        for t in range(blk - SLAG, blk):
            scat_desc(t).wait()
            if with_deg:
                ones_desc(t).wait()
        plsc.subcore_barrier()

        base = sid * rows_per_sub
        pltpu.sync_copy(acc.at[pl.ds(base, rows_per_sub)],
                        p_out.at[cid, pl.ds(base, rows_per_sub)])
        if with_deg:
            pltpu.sync_copy(dacc.at[pl.ds(base, rows_per_sub)],
                            deg_out.at[cid, pl.ds(base, rows_per_sub)])

    return pl.kernel(body, out_type=out_type, mesh=mesh,
                     scratch_types=scratch, interpret=interpret)


def _rmatT(a, w):
    return lax.dot_general(a, w, (((1,), (1,)), ((), ())),
                           preferred_element_type=jnp.float32)


def _layer_block(p_ref, h_ref, deg_ref, hcat_ref, post_ref, self_ref, b_ref):
    ssum = p_ref[0] + p_ref[1]
    hv = h_ref[...]
    deg = deg_ref[0] + deg_ref[1]
    base = jnp.where(deg > 0.0, ssum / jnp.maximum(deg, 1.0), hv)
    hashed = jnp.dot(base, hcat_ref[...], preferred_element_type=jnp.float32)
    out = _rmatT(hashed, post_ref[...]) + _rmatT(hv, self_ref[...])
    out = out + b_ref[...]
    return jnp.where(out > 0.0, out, jnp.exp(out) - 1.0)


def _dense_layer(p, h, deg2, hcat, post, slin, b_row, r=2000,
                 interpret=False):
    n, d = h.shape
    dh = hcat.shape[1]
    assert n % r == 0
    grid = (n // r,)

    def body(p_ref, h_ref, deg_ref, hcat_ref, post_ref, self_ref, b_ref, o_ref):
        o_ref[...] = _layer_block(p_ref, h_ref, deg_ref, hcat_ref, post_ref,
                                  self_ref, b_ref)

    return pl.pallas_call(
        body,
        grid=grid,
        in_specs=[
            pl.BlockSpec((NCORE, r, d), lambda i: (0, i, 0)),
            pl.BlockSpec((r, d), lambda i: (i, 0)),
            pl.BlockSpec((NCORE, r, 1), lambda i: (0, i, 0)),
            pl.BlockSpec((d, dh), lambda i: (0, 0)),
            pl.BlockSpec((d, dh), lambda i: (0, 0)),
            pl.BlockSpec((d, d), lambda i: (0, 0)),
            pl.BlockSpec((1, d), lambda i: (0, 0)),
        ],
        out_specs=pl.BlockSpec((r, d), lambda i: (i, 0)),
        out_shape=jax.ShapeDtypeStruct((n, d), jnp.float32),
        interpret=interpret,
    )(p, h, deg2, hcat, post, slin, b_row)


def _dense_pool_classify(p, h, deg2, hcat, post, slin, b_row, batch3,
                         cw, cb_row, n_graphs, interpret=False):
    n, d = h.shape
    dh = hcat.shape[1]
    pb, _, r = batch3.shape
    nc = cw.shape[0]
    assert n == pb * r
    grid = (pb,)

    def body(p_ref, h_ref, deg_ref, hcat_ref, post_ref, self_ref, b_ref,
             bat_ref, cw_ref, cb_ref, o_ref, gacc, cacc):
        i = pl.program_id(0)

        @pl.when(i == 0)
        def _():
            gacc[...] = jnp.zeros_like(gacc)
            cacc[...] = jnp.zeros_like(cacc)

        out = _layer_block(p_ref, h_ref, deg_ref, hcat_ref, post_ref,
                           self_ref, b_ref)

        ids = bat_ref[0]
        gids = lax.broadcasted_iota(jnp.int32, (n_graphs, r), 0)
        onehot = (ids == gids).astype(jnp.float32)
        gacc[...] += jnp.dot(onehot, out, preferred_element_type=jnp.float32)
        cacc[...] += jnp.sum(onehot, axis=1, keepdims=True)

        @pl.when(i == pb - 1)
        def _():
            gmean = gacc[...] / jnp.maximum(cacc[...], 1.0)
            o_ref[...] = _rmatT(gmean, cw_ref[...]) + cb_ref[...]

    return pl.pallas_call(
        body,
        grid=grid,
        in_specs=[
            pl.BlockSpec((NCORE, r, d), lambda i: (0, i, 0)),
            pl.BlockSpec((r, d), lambda i: (i, 0)),
            pl.BlockSpec((NCORE, r, 1), lambda i: (0, i, 0)),
            pl.BlockSpec((d, dh), lambda i: (0, 0)),
            pl.BlockSpec((d, dh), lambda i: (0, 0)),
            pl.BlockSpec((d, d), lambda i: (0, 0)),
            pl.BlockSpec((1, d), lambda i: (0, 0)),
            pl.BlockSpec((1, 1, r), lambda i: (i, 0, 0)),
            pl.BlockSpec((nc, d), lambda i: (0, 0)),
            pl.BlockSpec((1, nc), lambda i: (0, 0)),
        ],
        out_specs=pl.BlockSpec((n_graphs, nc), lambda i: (0, 0)),
        out_shape=jax.ShapeDtypeStruct((n_graphs, nc), jnp.float32),
        scratch_shapes=[
            pltpu.VMEM((n_graphs, d), jnp.float32),
            pltpu.VMEM((n_graphs, 1), jnp.float32),
        ],
        interpret=interpret,
    )(p, h, deg2, hcat, post, slin, b_row, batch3, cw, cb_row)


def kernel(x, edge_index, batch, H1, post1, self1, b1, H2, post2, self2, b2,
           cW, cb):
    n, d = x.shape
    e = edge_index.shape[1]
    g = 64
    nh = H1.shape[0]

    n_pad = -(-(n + 1) // (NSUB * 8 * 2)) * (NSUB * 8 * 2)
    if n_pad % 1024 != 0:
        n_pad = -(-n_pad // 1024) * 1024

    src = edge_index[0]
    dst = edge_index[1]
    blk = -(-e // (NW * KE))
    e_pad = NW * blk * KE
    dst_p = jnp.concatenate(
        [dst, jnp.zeros((e_pad - e,), dst.dtype)]).reshape(NW, blk, KE)
    src_p = jnp.concatenate(
        [src, jnp.full((e_pad - e,), n, src.dtype)]).reshape(NW, blk, KE)
    idx_p = jnp.stack([dst_p, src_p], axis=2)

    hcat1 = jnp.transpose(H1, (1, 0, 2)).reshape(d, nh * H1.shape[2])
    hcat2 = jnp.transpose(H2, (1, 0, 2)).reshape(d, nh * H2.shape[2])

    agg_deg = _build_agg(n_pad, d, blk, with_deg=True)
    agg = _build_agg(n_pad, d, blk, with_deg=False)

    p1, deg_p = agg_deg(x, idx_p)
    deg2 = deg_p[:, :, None]

    r = 2000
    h1 = _dense_layer(p1, x, deg2, hcat1, post1, self1, b1[None, :], r=r)

    (p2,) = agg(h1, idx_p)
    batch3 = batch.reshape(n // r, 1, r)
    return _dense_pool_classify(p2, h1, deg2, hcat2, post2, self2,
                                b2[None, :], batch3, cW, cb[None, :], g)


# speedup vs baseline: 13.7135x; 1.0015x over previous
import jax
import jax.numpy as jnp
from jax import lax
from jax.experimental import pallas as pl
from jax.experimental.pallas import tpu as pltpu
from jax.experimental.pallas import tpu_sc as plsc

NCORE = 2
NSUB = 16
NW = NCORE * NSUB
KE = 80
NRB = 4
NIB = 8
ILAG = 3
GLAG = 3
SLAG = 4


def _build_agg(n_pad, d, blk, with_deg, interpret=False):
    rows_per_sub = n_pad // NSUB
    zr = 16
    assert rows_per_sub % zr == 0 and n_pad % (NSUB * 8) == 0
    assert blk > SLAG + 1

    mesh = plsc.VectorSubcoreMesh(
        core_axis_name="c", subcore_axis_name="s",
        num_cores=NCORE, num_subcores=NSUB)

    out_type = [jax.ShapeDtypeStruct((NCORE, n_pad, d), jnp.float32)]
    if with_deg:
        out_type.append(jax.ShapeDtypeStruct((NCORE, n_pad), jnp.float32))

    scratch = [
        pltpu.VMEM((NIB, 2, KE), jnp.int32),
        pltpu.VMEM((NRB, KE, d), jnp.float32),
        pltpu.VMEM((KE,), jnp.float32),
        pltpu.VMEM((zr, d), jnp.float32),
        pltpu.VMEM_SHARED((n_pad, d), jnp.float32),
        pltpu.VMEM_SHARED((n_pad,), jnp.float32),
        pltpu.SemaphoreType.DMA,
        pltpu.SemaphoreType.DMA,
        pltpu.SemaphoreType.DMA,
        pltpu.SemaphoreType.DMA,
    ]

    def body(h_hbm, idx_hbm, *rest):
        if with_deg:
            p_out, deg_out = rest[0], rest[1]
            rest = rest[2:]
        else:
            p_out = rest[0]
            rest = rest[1:]
        ibuf, rows_v, ones_v, zbuf, acc, dacc, isem, gsem, ssem, osem = rest

        cid = lax.axis_index("c")
        sid = lax.axis_index("s")
        wid = sid * NCORE + cid

        z16 = jnp.zeros((16,), jnp.float32)
        o16 = jnp.ones((16,), jnp.float32)

        def idx_desc(b):
            return pltpu.make_async_copy(
                idx_hbm.at[wid, b], ibuf.at[lax.rem(b, NIB)], isem)

        def gather_desc(b):
            return pltpu.make_async_copy(
                h_hbm.at[ibuf.at[lax.rem(b, NIB), 0]],
                rows_v.at[lax.rem(b, NRB)], gsem)

        def scat_desc(b):
            return pltpu.make_async_copy(
                rows_v.at[lax.rem(b, NRB)],
                acc.at[ibuf.at[lax.rem(b, NIB), 1]], ssem)

        def ones_desc(b):
            return pltpu.make_async_copy(
                ones_v, dacc.at[ibuf.at[lax.rem(b, NIB), 1]], osem)

        for t in range(ILAG):
            idx_desc(t).start()

        def fill_row(i, _):
            for j in range(d // 16):
                zbuf[i, pl.ds(j * 16, 16)] = z16
            return 0
        lax.fori_loop(0, zr, fill_row, 0)

        def fill_ones(i, _):
            ones_v[pl.ds(i * 16, 16)] = o16
            return 0
        lax.fori_loop(0, KE // 16, fill_ones, 0)

        def clear_blk(k, _):
            pltpu.async_copy(
                zbuf, acc.at[pl.ds(sid * rows_per_sub + k * zr, zr)], gsem)
            return 0
        lax.fori_loop(0, rows_per_sub // zr, clear_blk, 0)
        if with_deg:
            def clear_deg(k, _):
                pltpu.async_copy(
                    zbuf.at[0],
                    dacc.at[pl.ds(sid * rows_per_sub + k * d, d)], osem)
                return 0
            lax.fori_loop(0, rows_per_sub // d, clear_deg, 0)

        def drain_blk(k, _):
            pltpu.make_async_copy(
                zbuf, acc.at[pl.ds(sid * rows_per_sub + k * zr, zr)],
                gsem).wait()
            return 0
        lax.fori_loop(0, rows_per_sub // zr, drain_blk, 0)
        if with_deg:
            def drain_deg(k, _):
                pltpu.make_async_copy(
                    zbuf.at[0],
                    dacc.at[pl.ds(sid * rows_per_sub + k * d, d)],
                    osem).wait()
                return 0
            lax.fori_loop(0, rows_per_sub // d, drain_deg, 0)
        plsc.subcore_barrier()

        def edge_blk(b, _):
            @pl.when(b >= SLAG)
            def _():
                scat_desc(b - SLAG).wait()
                if with_deg:
                    ones_desc(b - SLAG).wait()
            idx_desc(b).wait()
            gather_desc(b).start()

            @pl.when(b >= GLAG)
            def _():
                gather_desc(b - GLAG).wait()
                scat_desc(b - GLAG).start(add=True)
                if with_deg:
                    ones_desc(b - GLAG).start(add=True)

            @pl.when(b + ILAG < blk)
            def _():
                idx_desc(b + ILAG).start()
            return 0
        lax.fori_loop(0, blk, edge_blk, 0)

        for t in range(blk - GLAG, blk):
            gather_desc(t).wait()
            scat_desc(t).start(add=True)
            if with_deg:
                ones_desc(t).start(add=True)
        for t in range(blk - SLAG, blk):
            scat_desc(t).wait()
            if with_deg:
                ones_desc(t).wait()
        plsc.subcore_barrier()

        base = sid * rows_per_sub
        pltpu.sync_copy(acc.at[pl.ds(base, rows_per_sub)],
                        p_out.at[cid, pl.ds(base, rows_per_sub)])
        if with_deg:
            pltpu.sync_copy(dacc.at[pl.ds(base, rows_per_sub)],
                            deg_out.at[cid, pl.ds(base, rows_per_sub)])

    return pl.kernel(body, out_type=out_type, mesh=mesh,
                     scratch_types=scratch, interpret=interpret)


def _rmatT(a, w):
    return lax.dot_general(a, w, (((1,), (1,)), ((), ())),
                           preferred_element_type=jnp.float32)


def _layer_block(p_ref, h_ref, deg_ref, hcat_ref, post_ref, self_ref, b_ref):
    ssum = p_ref[0] + p_ref[1]
    hv = h_ref[...]
    deg = deg_ref[0] + deg_ref[1]
    base = jnp.where(deg > 0.0, ssum / jnp.maximum(deg, 1.0), hv)
    hashed = jnp.dot(base, hcat_ref[...], preferred_element_type=jnp.float32)
    out = _rmatT(hashed, post_ref[...]) + _rmatT(hv, self_ref[...])
    out = out + b_ref[...]
    return jnp.where(out > 0.0, out, jnp.exp(out) - 1.0)


def _dense_layer(p, h, deg2, hcat, post, slin, b_row, r=2000,
                 interpret=False):
    n, d = h.shape
    dh = hcat.shape[1]
    assert n % r == 0
    grid = (n // r,)

    def body(p_ref, h_ref, deg_ref, hcat_ref, post_ref, self_ref, b_ref, o_ref):
        o_ref[...] = _layer_block(p_ref, h_ref, deg_ref, hcat_ref, post_ref,
                                  self_ref, b_ref)

    return pl.pallas_call(
        body,
        grid=grid,
        in_specs=[
            pl.BlockSpec((NCORE, r, d), lambda i: (0, i, 0)),
            pl.BlockSpec((r, d), lambda i: (i, 0)),
            pl.BlockSpec((NCORE, r, 1), lambda i: (0, i, 0)),
            pl.BlockSpec((d, dh), lambda i: (0, 0)),
            pl.BlockSpec((d, dh), lambda i: (0, 0)),
            pl.BlockSpec((d, d), lambda i: (0, 0)),
            pl.BlockSpec((1, d), lambda i: (0, 0)),
        ],
        out_specs=pl.BlockSpec((r, d), lambda i: (i, 0)),
        out_shape=jax.ShapeDtypeStruct((n, d), jnp.float32),
        interpret=interpret,
    )(p, h, deg2, hcat, post, slin, b_row)


def _dense_pool_classify(p, h, deg2, hcat, post, slin, b_row, batch3,
                         cw, cb_row, n_graphs, interpret=False):
    n, d = h.shape
    dh = hcat.shape[1]
    pb, _, r = batch3.shape
    nc = cw.shape[0]
    assert n == pb * r
    grid = (pb,)

    def body(p_ref, h_ref, deg_ref, hcat_ref, post_ref, self_ref, b_ref,
             bat_ref, cw_ref, cb_ref, o_ref, gacc, cacc):
        i = pl.program_id(0)

        @pl.when(i == 0)
        def _():
            gacc[...] = jnp.zeros_like(gacc)
            cacc[...] = jnp.zeros_like(cacc)

        out = _layer_block(p_ref, h_ref, deg_ref, hcat_ref, post_ref,
                           self_ref, b_ref)

        ids = bat_ref[0]
        gids = lax.broadcasted_iota(jnp.int32, (n_graphs, r), 0)
        onehot = (ids == gids).astype(jnp.float32)
        gacc[...] += jnp.dot(onehot, out, preferred_element_type=jnp.float32)
        cacc[...] += jnp.sum(onehot, axis=1, keepdims=True)

        @pl.when(i == pb - 1)
        def _():
            gmean = gacc[...] / jnp.maximum(cacc[...], 1.0)
            o_ref[...] = _rmatT(gmean, cw_ref[...]) + cb_ref[...]

    return pl.pallas_call(
        body,
        grid=grid,
        in_specs=[
            pl.BlockSpec((NCORE, r, d), lambda i: (0, i, 0)),
            pl.BlockSpec((r, d), lambda i: (i, 0)),
            pl.BlockSpec((NCORE, r, 1), lambda i: (0, i, 0)),
            pl.BlockSpec((d, dh), lambda i: (0, 0)),
            pl.BlockSpec((d, dh), lambda i: (0, 0)),
            pl.BlockSpec((d, d), lambda i: (0, 0)),
            pl.BlockSpec((1, d), lambda i: (0, 0)),
            pl.BlockSpec((1, 1, r), lambda i: (i, 0, 0)),
            pl.BlockSpec((nc, d), lambda i: (0, 0)),
            pl.BlockSpec((1, nc), lambda i: (0, 0)),
        ],
        out_specs=pl.BlockSpec((n_graphs, nc), lambda i: (0, 0)),
        out_shape=jax.ShapeDtypeStruct((n_graphs, nc), jnp.float32),
        scratch_shapes=[
            pltpu.VMEM((n_graphs, d), jnp.float32),
            pltpu.VMEM((n_graphs, 1), jnp.float32),
        ],
        interpret=interpret,
    )(p, h, deg2, hcat, post, slin, b_row, batch3, cw, cb_row)


def kernel(x, edge_index, batch, H1, post1, self1, b1, H2, post2, self2, b2,
           cW, cb):
    n, d = x.shape
    e = edge_index.shape[1]
    g = 64
    nh = H1.shape[0]

    n_pad = -(-(n + 1) // (NSUB * 8 * 2)) * (NSUB * 8 * 2)
    if n_pad % 1024 != 0:
        n_pad = -(-n_pad // 1024) * 1024

    src = edge_index[0]
    dst = edge_index[1]
    blk = -(-e // (NW * KE))
    e_pad = NW * blk * KE
    dst_p = jnp.concatenate(
        [dst, jnp.zeros((e_pad - e,), dst.dtype)]).reshape(NW, blk, KE)
    src_p = jnp.concatenate(
        [src, jnp.full((e_pad - e,), n, src.dtype)]).reshape(NW, blk, KE)
    idx_p = jnp.stack([dst_p, src_p], axis=2)

    hcat1 = jnp.transpose(H1, (1, 0, 2)).reshape(d, nh * H1.shape[2])
    hcat2 = jnp.transpose(H2, (1, 0, 2)).reshape(d, nh * H2.shape[2])

    agg_deg = _build_agg(n_pad, d, blk, with_deg=True)
    agg = _build_agg(n_pad, d, blk, with_deg=False)

    p1, deg_p = agg_deg(x, idx_p)
    deg2 = deg_p[:, :, None]

    r = 2000
    h1 = _dense_layer(p1, x, deg2, hcat1, post1, self1, b1[None, :], r=r)

    (p2,) = agg(h1, idx_p)
    batch3 = batch.reshape(n // r, 1, r)
    return _dense_pool_classify(p2, h1, deg2, hcat2, post2, self2,
                                b2[None, :], batch3, cW, cb[None, :], g)
